# baseline (speedup 1.0000x reference)
import jax
import jax.numpy as jnp
from jax.experimental import pallas as pl
from jax.experimental.pallas import tpu as pltpu

_CHUNK = 1000
_NBUF = 4


def _pipelined_kernel(x_hbm, wt_ref, b_ref, o_hbm, xbuf, ybuf, in_sems, out_sems):
    n = x_hbm.shape[0]
    nchunks = n // _CHUNK

    def start_in(i, s):
        pltpu.async_copy(
            x_hbm.at[pl.ds(i * _CHUNK, _CHUNK), :], xbuf.at[s], in_sems.at[s]
        )

    def wait_in(i, s):
        pltpu.make_async_copy(
            x_hbm.at[pl.ds(i * _CHUNK, _CHUNK), :], xbuf.at[s], in_sems.at[s]
        ).wait()

    def start_out(i, s):
        pltpu.async_copy(
            ybuf.at[s], o_hbm.at[pl.ds(i * _CHUNK, _CHUNK), :], out_sems.at[s],
            priority=1,
        )

    def wait_out(i, s):
        pltpu.make_async_copy(
            ybuf.at[s], o_hbm.at[pl.ds(i * _CHUNK, _CHUNK), :], out_sems.at[s]
        ).wait()

    for s in range(min(_NBUF, nchunks)):
        start_in(s, s)
    wt = wt_ref[...]
    bias = b_ref[...]
    for i in range(nchunks):
        s = i % _NBUF
        wait_in(i, s)
        if i >= _NBUF:
            wait_out(i - _NBUF, s)
        acc = jnp.dot(xbuf[s], wt, preferred_element_type=jnp.float32)
        ybuf[s] = jnp.maximum(acc + bias, 0.0)
        start_out(i, s)
        if i + _NBUF < nchunks:
            start_in(i + _NBUF, s)
    for i in range(max(0, nchunks - _NBUF), nchunks):
        wait_out(i, i % _NBUF)


def kernel(feats, edge_index, W, b, agg_weight):
    del edge_index, agg_weight
    n, in_feats = feats.shape
    out_feats = W.shape[0]
    wt = W.T
    b2 = b.reshape(1, out_feats)
    return pl.pallas_call(
        _pipelined_kernel,
        in_specs=[
            pl.BlockSpec(memory_space=pl.ANY),
            pl.BlockSpec(memory_space=pltpu.MemorySpace.VMEM),
            pl.BlockSpec(memory_space=pltpu.MemorySpace.VMEM),
        ],
        out_specs=pl.BlockSpec(memory_space=pl.ANY),
        out_shape=jax.ShapeDtypeStruct((n, out_feats), jnp.float32),
        scratch_shapes=[
            pltpu.VMEM((_NBUF, _CHUNK, in_feats), jnp.float32),
            pltpu.VMEM((_NBUF, _CHUNK, out_feats), jnp.float32),
            pltpu.SemaphoreType.DMA((_NBUF,)),
            pltpu.SemaphoreType.DMA((_NBUF,)),
        ],
    )(feats, wt, b2)


# speedup vs baseline: 1.0330x; 1.0330x over previous
import jax
import jax.numpy as jnp
from jax.experimental import pallas as pl
from jax.experimental.pallas import tpu as pltpu

_CHUNK = 1000


def _pipelined_kernel(x_hbm, wt_ref, b_ref, o_hbm, xbuf, ybuf, in_sems, out_sems):
    n = x_hbm.shape[0]
    nchunks = n // _CHUNK

    def in_cp(i):
        return pltpu.make_async_copy(
            x_hbm.at[pl.ds(i * _CHUNK, _CHUNK), :], xbuf.at[i], in_sems.at[i]
        )

    def out_cp(i):
        return pltpu.make_async_copy(
            ybuf.at[i], o_hbm.at[pl.ds(i * _CHUNK, _CHUNK), :], out_sems.at[i]
        )

    for i in range(nchunks):
        in_cp(i).start()
    wt = wt_ref[...]
    bias = b_ref[...]
    for i in range(nchunks):
        in_cp(i).wait()
        acc = jnp.dot(xbuf[i], wt, preferred_element_type=jnp.float32)
        ybuf[i] = jnp.maximum(acc + bias, 0.0)
        out_cp(i).start()
    for i in range(nchunks):
        out_cp(i).wait()


def kernel(feats, edge_index, W, b, agg_weight):
    del edge_index, agg_weight
    n, in_feats = feats.shape
    out_feats = W.shape[0]
    wt = W.T
    b2 = b.reshape(1, out_feats)
    nchunks = n // _CHUNK
    return pl.pallas_call(
        _pipelined_kernel,
        in_specs=[
            pl.BlockSpec(memory_space=pl.ANY),
            pl.BlockSpec(memory_space=pltpu.MemorySpace.VMEM),
            pl.BlockSpec(memory_space=pltpu.MemorySpace.VMEM),
        ],
        out_specs=pl.BlockSpec(memory_space=pl.ANY),
        out_shape=jax.ShapeDtypeStruct((n, out_feats), jnp.float32),
        scratch_shapes=[
            pltpu.VMEM((nchunks, _CHUNK, in_feats), jnp.float32),
            pltpu.VMEM((nchunks, _CHUNK, out_feats), jnp.float32),
            pltpu.SemaphoreType.DMA((nchunks,)),
            pltpu.SemaphoreType.DMA((nchunks,)),
        ],
    )(feats, wt, b2)


# speedup vs baseline: 1.6208x; 1.5690x over previous
import jax
import jax.numpy as jnp
from jax.experimental import pallas as pl
from jax.experimental.pallas import tpu as pltpu

_CHUNK = 2000


def _pipelined_kernel(x_hbm, w_ref, b_ref, o_hbm, xbuf, ybuf, in_sems, out_sems):
    n = x_hbm.shape[0]
    nchunks = n // _CHUNK

    def in_cp(i):
        return pltpu.make_async_copy(
            x_hbm.at[pl.ds(i * _CHUNK, _CHUNK), :], xbuf.at[i], in_sems.at[i]
        )

    def out_cp(i):
        return pltpu.make_async_copy(
            ybuf.at[i], o_hbm.at[pl.ds(i * _CHUNK, _CHUNK), :], out_sems.at[i]
        )

    for i in range(nchunks):
        in_cp(i).start()
    w = w_ref[...]
    bias = b_ref[...]
    for i in range(nchunks):
        in_cp(i).wait()
        acc = jax.lax.dot_general(
            xbuf[i], w, (((1,), (1,)), ((), ())),
            preferred_element_type=jnp.float32,
        )
        ybuf[i] = jnp.maximum(acc + bias, 0.0)
        out_cp(i).start()
    for i in range(nchunks):
        out_cp(i).wait()


def kernel(feats, edge_index, W, b, agg_weight):
    del edge_index, agg_weight
    n, in_feats = feats.shape
    out_feats = W.shape[0]
    b2 = b.reshape(1, out_feats)
    nchunks = n // _CHUNK
    return pl.pallas_call(
        _pipelined_kernel,
        in_specs=[
            pl.BlockSpec(memory_space=pl.ANY),
            pl.BlockSpec(memory_space=pltpu.MemorySpace.VMEM),
            pl.BlockSpec(memory_space=pltpu.MemorySpace.VMEM),
        ],
        out_specs=pl.BlockSpec(memory_space=pl.ANY),
        out_shape=jax.ShapeDtypeStruct((n, out_feats), jnp.float32),
        scratch_shapes=[
            pltpu.VMEM((nchunks, _CHUNK, in_feats), jnp.float32),
            pltpu.VMEM((nchunks, _CHUNK, out_feats), jnp.float32),
            pltpu.SemaphoreType.DMA((nchunks,)),
            pltpu.SemaphoreType.DMA((nchunks,)),
        ],
    )(feats, W, b2)
